# baseline (speedup 1.0000x reference)
import functools

import jax
import jax.numpy as jnp
import numpy as np
from jax import lax
from jax.experimental import pallas as pl
from jax.experimental.pallas import tpu as pltpu
from jax.experimental.pallas import tpu_sc as plsc

_NBUF = 8
_LA = 4


@functools.lru_cache(maxsize=None)
def _mix_constants(bs: int):
    with jax.ensure_compile_time_eval():
        key = jax.random.key(42)
        kp, kr, kc = jax.random.split(key, 3)
        perm = jax.random.permutation(kp, bs)
        keep = jax.random.uniform(kr, (bs,)) < 1.0
        perm = jnp.where(keep, perm, jnp.arange(bs))
        coeffs = jax.random.beta(kc, 0.4, 0.4, (bs,)).astype(jnp.float32)
    return np.asarray(perm, dtype=np.int32), np.asarray(coeffs, dtype=np.float32)


@functools.lru_cache(maxsize=None)
def _schedule(bs: int):
    perm, coeffs = _mix_constants(bs)
    visited = np.zeros(bs, dtype=bool)
    order, ishead, isend = [], [], []
    for s in range(bs):
        if visited[s]:
            continue
        i = s
        first = True
        while not visited[i]:
            visited[i] = True
            order.append(i)
            ishead.append(1 if first else 0)
            isend.append(0)
            first = False
            i = int(perm[i])
        isend[-1] = 1
    order = np.asarray(order, dtype=np.int32)
    ishead = np.asarray(ishead, dtype=np.int32)
    isend = np.asarray(isend, dtype=np.int32)

    ld = np.concatenate([order, np.zeros(1 + _LA, np.int32)])
    oidx = np.concatenate([order[:1], order])
    e = np.concatenate([np.zeros(1, np.int32), isend])
    hd = np.concatenate([ishead, np.zeros(1, np.int32)])
    cs = coeffs[oidx]
    return ld, oidx, e, hd, cs


def _make_tc_body(bs: int):
    def _body(ld, oidx, e, hd, cs, xin, out, ring, obuf, head, insem, outsem):
        t = pl.program_id(0)

        @pl.when(t == 0)
        def _():
            for j in range(_LA):
                pltpu.make_async_copy(
                    xin.at[ld[j]], ring.at[j], insem.at[j]).start()

        @pl.when(t + _LA < bs)
        def _():
            slot = lax.rem(t + _LA, _NBUF)
            pltpu.make_async_copy(
                xin.at[ld[t + _LA]], ring.at[slot], insem.at[slot]).start()

        @pl.when(t < bs)
        def _():
            slot = lax.rem(t, _NBUF)
            pltpu.make_async_copy(
                xin.at[ld[t]], ring.at[slot], insem.at[slot]).wait()

        @pl.when(t > 0)
        def _():
            u = t - 1
            c = cs[t]
            cur = lax.rem(t, _NBUF)
            prv = lax.rem(u, _NBUF)
            ob = lax.rem(u, 2)

            @pl.when(u >= 2)
            def _():
                pltpu.make_async_copy(
                    obuf.at[ob], out.at[oidx[t - 2]], outsem.at[ob]).wait()

            @pl.when(e[t] == 0)
            def _():
                obuf[ob] = c * ring[prv] + (1.0 - c) * ring[cur]

            @pl.when(e[t] == 1)
            def _():
                obuf[ob] = c * ring[prv] + (1.0 - c) * head[...]

            pltpu.make_async_copy(
                obuf.at[ob], out.at[oidx[t]], outsem.at[ob]).start()

        @pl.when(hd[t] == 1)
        def _():
            head[...] = ring[lax.rem(t, _NBUF)]

        @pl.when(t == bs)
        def _():
            pltpu.make_async_copy(
                obuf.at[(bs - 2) % 2], out.at[oidx[bs - 1]],
                outsem.at[(bs - 2) % 2]).wait()
            pltpu.make_async_copy(
                obuf.at[(bs - 1) % 2], out.at[oidx[bs]],
                outsem.at[(bs - 1) % 2]).wait()

    return _body


def _mix_x(xr):
    bs, r, _ = xr.shape
    ld, oidx, e, hd, cs = _schedule(bs)
    grid_spec = pltpu.PrefetchScalarGridSpec(
        num_scalar_prefetch=5,
        grid=(bs + 1,),
        in_specs=[pl.BlockSpec(memory_space=pl.ANY)],
        out_specs=pl.BlockSpec(memory_space=pl.ANY),
        scratch_shapes=[
            pltpu.VMEM((_NBUF, r, 128), jnp.float32),
            pltpu.VMEM((2, r, 128), jnp.float32),
            pltpu.VMEM((r, 128), jnp.float32),
            pltpu.SemaphoreType.DMA((_NBUF,)),
            pltpu.SemaphoreType.DMA((2,)),
        ],
    )
    return pl.pallas_call(
        _make_tc_body(bs),
        grid_spec=grid_spec,
        out_shape=jax.ShapeDtypeStruct((bs, r, 128), jnp.float32),
        compiler_params=pltpu.CompilerParams(
            dimension_semantics=("arbitrary",),
        ),
    )(jnp.asarray(ld), jnp.asarray(oidx), jnp.asarray(e), jnp.asarray(hd),
      jnp.asarray(cs), xr)


def _sc_label_body(y2_hbm, ya_hbm, pidx_hbm, c2_hbm, yo_hbm, zo_hbm,
                   idx_v, cv, y2s, y2p, zas, zap, yov, zov, sem):
    ncp = y2s.shape[1]
    wid = lax.axis_index("s") * 2 + lax.axis_index("c")
    rows = idx_v.shape[0]
    base = wid * rows
    pltpu.sync_copy(pidx_hbm.at[pl.ds(base, rows)], idx_v)
    pltpu.sync_copy(c2_hbm.at[pl.ds(base, rows)], cv)
    pltpu.sync_copy(y2_hbm.at[pl.ds(base, rows)], y2s)
    pltpu.sync_copy(ya_hbm.at[pl.ds(base, rows)], zas)
    pltpu.async_copy(y2_hbm.at[idx_v], y2p, sem).wait()
    pltpu.async_copy(ya_hbm.at[idx_v], zap, sem).wait()
    for i in range(rows):
        cvec = cv[i]

        def chunk(j, carry, i=i, cvec=cvec):
            sl = pl.ds(j * 16, 16)
            a = y2s[i, sl]
            b = y2p[i, sl]
            ym = b + cvec * (a - b)
            yov[i, sl] = ym
            z = jnp.maximum(zas[i, sl], zap[i, sl]) - ym
            zov[i, sl] = jnp.minimum(jnp.maximum(z, 0.0), 1.0)
            return carry

        lax.fori_loop(0, ncp // 16, chunk, 0)
    pltpu.sync_copy(yov, yo_hbm.at[pl.ds(base, rows)])
    pltpu.sync_copy(zov, zo_hbm.at[pl.ds(base, rows)])


def _mix_labels(y2, ya2, perm, coeffs):
    bs, ncp = y2.shape
    nw = 32
    rows = bs // nw
    mesh = plsc.VectorSubcoreMesh(core_axis_name="c", subcore_axis_name="s")
    f32 = jnp.float32
    k = functools.partial(
        pl.kernel,
        mesh=mesh,
        out_type=[
            jax.ShapeDtypeStruct((bs, ncp), f32),
            jax.ShapeDtypeStruct((bs, ncp), f32),
        ],
        scratch_types=[
            pltpu.VMEM((rows,), jnp.int32),
            pltpu.VMEM((rows, 16), f32),
            pltpu.VMEM((rows, ncp), f32),
            pltpu.VMEM((rows, ncp), f32),
            pltpu.VMEM((rows, ncp), f32),
            pltpu.VMEM((rows, ncp), f32),
            pltpu.VMEM((rows, ncp), f32),
            pltpu.VMEM((rows, ncp), f32),
            pltpu.SemaphoreType.DMA,
        ],
    )(_sc_label_body)
    c2 = jnp.broadcast_to(coeffs[:, None], (bs, 16))
    return k(y2, ya2, perm, c2)


def kernel(x, y, y_aux, w):
    bs = x.shape[0]
    n = int(np.prod(x.shape[1:]))
    assert n % 128 == 0
    xr = x.reshape(bs, n // 128, 128)

    nc = y.shape[1]
    pad = (-(nc + 1)) % 1024
    y2 = jnp.concatenate(
        [y, w[:, None], jnp.zeros((bs, pad), jnp.float32)], axis=1)
    ncp = nc + 1 + pad
    ya2 = jnp.pad(y_aux, ((0, 0), (0, ncp - nc)))

    perm, coeffs = _mix_constants(bs)
    yo, zo = _mix_labels(y2, ya2, jnp.asarray(perm), jnp.asarray(coeffs))
    xo = _mix_x(xr)

    x_mix = xo.reshape(x.shape)
    y_mix = yo[:, :nc]
    w_mix = yo[:, nc]
    ya_mix = zo[:, :nc]
    return (x_mix, y_mix, ya_mix, w_mix)


# speedup vs baseline: 1.0011x; 1.0011x over previous
import functools

import jax
import jax.numpy as jnp
import numpy as np
from jax import lax
from jax.experimental import pallas as pl
from jax.experimental.pallas import tpu as pltpu
from jax.experimental.pallas import tpu_sc as plsc

_NBUF = 12
_LA = 6


@functools.lru_cache(maxsize=None)
def _mix_constants(bs: int):
    with jax.ensure_compile_time_eval():
        key = jax.random.key(42)
        kp, kr, kc = jax.random.split(key, 3)
        perm = jax.random.permutation(kp, bs)
        keep = jax.random.uniform(kr, (bs,)) < 1.0
        perm = jnp.where(keep, perm, jnp.arange(bs))
        coeffs = jax.random.beta(kc, 0.4, 0.4, (bs,)).astype(jnp.float32)
    return np.asarray(perm, dtype=np.int32), np.asarray(coeffs, dtype=np.float32)


@functools.lru_cache(maxsize=None)
def _schedule(bs: int):
    perm, coeffs = _mix_constants(bs)
    visited = np.zeros(bs, dtype=bool)
    order, ishead, isend = [], [], []
    for s in range(bs):
        if visited[s]:
            continue
        i = s
        first = True
        while not visited[i]:
            visited[i] = True
            order.append(i)
            ishead.append(1 if first else 0)
            isend.append(0)
            first = False
            i = int(perm[i])
        isend[-1] = 1
    order = np.asarray(order, dtype=np.int32)
    ishead = np.asarray(ishead, dtype=np.int32)
    isend = np.asarray(isend, dtype=np.int32)

    ld = np.concatenate([order, np.zeros(1 + _LA, np.int32)])
    oidx = np.concatenate([order[:1], order])
    e = np.concatenate([np.zeros(1, np.int32), isend])
    hd = np.concatenate([ishead, np.zeros(1, np.int32)])
    cs = coeffs[oidx]
    return ld, oidx, e, hd, cs


def _make_tc_body(bs: int):
    def _body(ld, oidx, e, hd, cs, xin, out, ring, obuf, head, insem, outsem):
        t = pl.program_id(0)

        @pl.when(t == 0)
        def _():
            for j in range(_LA):
                pltpu.make_async_copy(
                    xin.at[ld[j]], ring.at[j], insem.at[j]).start()

        @pl.when(t + _LA < bs)
        def _():
            slot = lax.rem(t + _LA, _NBUF)
            pltpu.make_async_copy(
                xin.at[ld[t + _LA]], ring.at[slot], insem.at[slot]).start()

        @pl.when(t < bs)
        def _():
            slot = lax.rem(t, _NBUF)
            pltpu.make_async_copy(
                xin.at[ld[t]], ring.at[slot], insem.at[slot]).wait()

        @pl.when(t > 0)
        def _():
            u = t - 1
            c = cs[t]
            cur = lax.rem(t, _NBUF)
            prv = lax.rem(u, _NBUF)
            ob = lax.rem(u, 2)

            @pl.when(u >= 2)
            def _():
                pltpu.make_async_copy(
                    obuf.at[ob], out.at[oidx[t - 2]], outsem.at[ob]).wait()

            @pl.when(e[t] == 0)
            def _():
                obuf[ob] = c * ring[prv] + (1.0 - c) * ring[cur]

            @pl.when(e[t] == 1)
            def _():
                obuf[ob] = c * ring[prv] + (1.0 - c) * head[...]

            pltpu.make_async_copy(
                obuf.at[ob], out.at[oidx[t]], outsem.at[ob]).start()

        @pl.when(hd[t] == 1)
        def _():
            head[...] = ring[lax.rem(t, _NBUF)]

        @pl.when(t == bs)
        def _():
            pltpu.make_async_copy(
                obuf.at[(bs - 2) % 2], out.at[oidx[bs - 1]],
                outsem.at[(bs - 2) % 2]).wait()
            pltpu.make_async_copy(
                obuf.at[(bs - 1) % 2], out.at[oidx[bs]],
                outsem.at[(bs - 1) % 2]).wait()

    return _body


def _mix_x(xr):
    bs, r, _ = xr.shape
    ld, oidx, e, hd, cs = _schedule(bs)
    grid_spec = pltpu.PrefetchScalarGridSpec(
        num_scalar_prefetch=5,
        grid=(bs + 1,),
        in_specs=[pl.BlockSpec(memory_space=pl.ANY)],
        out_specs=pl.BlockSpec(memory_space=pl.ANY),
        scratch_shapes=[
            pltpu.VMEM((_NBUF, r, 128), jnp.float32),
            pltpu.VMEM((2, r, 128), jnp.float32),
            pltpu.VMEM((r, 128), jnp.float32),
            pltpu.SemaphoreType.DMA((_NBUF,)),
            pltpu.SemaphoreType.DMA((2,)),
        ],
    )
    return pl.pallas_call(
        _make_tc_body(bs),
        grid_spec=grid_spec,
        out_shape=jax.ShapeDtypeStruct((bs, r, 128), jnp.float32),
        compiler_params=pltpu.CompilerParams(
            dimension_semantics=("arbitrary",),
        ),
    )(jnp.asarray(ld), jnp.asarray(oidx), jnp.asarray(e), jnp.asarray(hd),
      jnp.asarray(cs), xr)


def _sc_label_body(y2_hbm, ya_hbm, pidx_hbm, c2_hbm, yo_hbm, zo_hbm,
                   idx_v, cv, y2s, y2p, zas, zap, yov, zov, sem):
    ncp = y2s.shape[1]
    wid = lax.axis_index("s") * 2 + lax.axis_index("c")
    rows = idx_v.shape[0]
    base = wid * rows
    pltpu.sync_copy(pidx_hbm.at[pl.ds(base, rows)], idx_v)
    pltpu.sync_copy(c2_hbm.at[pl.ds(base, rows)], cv)
    pltpu.sync_copy(y2_hbm.at[pl.ds(base, rows)], y2s)
    pltpu.sync_copy(ya_hbm.at[pl.ds(base, rows)], zas)
    pltpu.async_copy(y2_hbm.at[idx_v], y2p, sem).wait()
    pltpu.async_copy(ya_hbm.at[idx_v], zap, sem).wait()
    for i in range(rows):
        cvec = cv[i]

        def chunk(j, carry, i=i, cvec=cvec):
            sl = pl.ds(j * 16, 16)
            a = y2s[i, sl]
            b = y2p[i, sl]
            ym = b + cvec * (a - b)
            yov[i, sl] = ym
            z = jnp.maximum(zas[i, sl], zap[i, sl]) - ym
            zov[i, sl] = jnp.minimum(jnp.maximum(z, 0.0), 1.0)
            return carry

        lax.fori_loop(0, ncp // 16, chunk, 0)
    pltpu.sync_copy(yov, yo_hbm.at[pl.ds(base, rows)])
    pltpu.sync_copy(zov, zo_hbm.at[pl.ds(base, rows)])


def _mix_labels(y2, ya2, perm, coeffs):
    bs, ncp = y2.shape
    nw = 32
    rows = bs // nw
    mesh = plsc.VectorSubcoreMesh(core_axis_name="c", subcore_axis_name="s")
    f32 = jnp.float32
    k = functools.partial(
        pl.kernel,
        mesh=mesh,
        out_type=[
            jax.ShapeDtypeStruct((bs, ncp), f32),
            jax.ShapeDtypeStruct((bs, ncp), f32),
        ],
        scratch_types=[
            pltpu.VMEM((rows,), jnp.int32),
            pltpu.VMEM((rows, 16), f32),
            pltpu.VMEM((rows, ncp), f32),
            pltpu.VMEM((rows, ncp), f32),
            pltpu.VMEM((rows, ncp), f32),
            pltpu.VMEM((rows, ncp), f32),
            pltpu.VMEM((rows, ncp), f32),
            pltpu.VMEM((rows, ncp), f32),
            pltpu.SemaphoreType.DMA,
        ],
    )(_sc_label_body)
    c2 = jnp.broadcast_to(coeffs[:, None], (bs, 16))
    return k(y2, ya2, perm, c2)


def kernel(x, y, y_aux, w):
    bs = x.shape[0]
    n = int(np.prod(x.shape[1:]))
    assert n % 128 == 0
    xr = x.reshape(bs, n // 128, 128)

    nc = y.shape[1]
    pad = (-(nc + 1)) % 1024
    y2 = jnp.concatenate(
        [y, w[:, None], jnp.zeros((bs, pad), jnp.float32)], axis=1)
    ncp = nc + 1 + pad
    ya2 = jnp.pad(y_aux, ((0, 0), (0, ncp - nc)))

    perm, coeffs = _mix_constants(bs)
    yo, zo = _mix_labels(y2, ya2, jnp.asarray(perm), jnp.asarray(coeffs))
    xo = _mix_x(xr)

    x_mix = xo.reshape(x.shape)
    y_mix = yo[:, :nc]
    w_mix = yo[:, nc]
    ya_mix = zo[:, :nc]
    return (x_mix, y_mix, ya_mix, w_mix)


# speedup vs baseline: 1.0029x; 1.0018x over previous
import functools

import jax
import jax.numpy as jnp
import numpy as np
from jax import lax
from jax.experimental import pallas as pl
from jax.experimental.pallas import tpu as pltpu
from jax.experimental.pallas import tpu_sc as plsc

_NBUF = 8
_LA = 4


@functools.lru_cache(maxsize=None)
def _mix_constants(bs: int):
    with jax.ensure_compile_time_eval():
        key = jax.random.key(42)
        kp, kr, kc = jax.random.split(key, 3)
        perm = jax.random.permutation(kp, bs)
        keep = jax.random.uniform(kr, (bs,)) < 1.0
        perm = jnp.where(keep, perm, jnp.arange(bs))
        coeffs = jax.random.beta(kc, 0.4, 0.4, (bs,)).astype(jnp.float32)
    return np.asarray(perm, dtype=np.int32), np.asarray(coeffs, dtype=np.float32)


@functools.lru_cache(maxsize=None)
def _schedule(bs: int):
    perm, coeffs = _mix_constants(bs)
    visited = np.zeros(bs, dtype=bool)
    order, ishead, isend = [], [], []
    for s in range(bs):
        if visited[s]:
            continue
        i = s
        first = True
        while not visited[i]:
            visited[i] = True
            order.append(i)
            ishead.append(1 if first else 0)
            isend.append(0)
            first = False
            i = int(perm[i])
        isend[-1] = 1
    order = np.asarray(order, dtype=np.int32)
    ishead = np.asarray(ishead, dtype=np.int32)
    isend = np.asarray(isend, dtype=np.int32)

    ld = np.concatenate([order, np.zeros(1 + _LA, np.int32)])
    oidx = np.concatenate([order[:1], order])
    e = np.concatenate([np.zeros(1, np.int32), isend])
    hd = np.concatenate([ishead, np.zeros(1, np.int32)])
    cs = coeffs[oidx]
    return ld, oidx, e, hd, cs


def _make_tc_body(bs: int):
    def _body(ld, oidx, e, hd, cs, xin, out, ring, obuf, head, insem, outsem):
        t = pl.program_id(0)

        @pl.when(t == 0)
        def _():
            for j in range(_LA):
                pltpu.make_async_copy(
                    xin.at[ld[j]], ring.at[j], insem.at[j]).start()

        @pl.when(t + _LA < bs)
        def _():
            slot = lax.rem(t + _LA, _NBUF)
            pltpu.make_async_copy(
                xin.at[ld[t + _LA]], ring.at[slot], insem.at[slot]).start()

        @pl.when(t < bs)
        def _():
            slot = lax.rem(t, _NBUF)
            pltpu.make_async_copy(
                xin.at[ld[t]], ring.at[slot], insem.at[slot]).wait()

        @pl.when(t > 0)
        def _():
            u = t - 1
            c = cs[t]
            cur = lax.rem(t, _NBUF)
            prv = lax.rem(u, _NBUF)
            ob = lax.rem(u, 2)

            @pl.when(u >= 2)
            def _():
                pltpu.make_async_copy(
                    obuf.at[ob], out.at[oidx[t - 2]], outsem.at[ob]).wait()

            @pl.when(e[t] == 0)
            def _():
                obuf[ob] = c * ring[prv] + (1.0 - c) * ring[cur]

            @pl.when(e[t] == 1)
            def _():
                obuf[ob] = c * ring[prv] + (1.0 - c) * head[...]

            pltpu.make_async_copy(
                obuf.at[ob], out.at[oidx[t]], outsem.at[ob]).start()

        @pl.when(hd[t] == 1)
        def _():
            head[...] = ring[lax.rem(t, _NBUF)]

        @pl.when(t == bs)
        def _():
            pltpu.make_async_copy(
                obuf.at[(bs - 2) % 2], out.at[oidx[bs - 1]],
                outsem.at[(bs - 2) % 2]).wait()
            pltpu.make_async_copy(
                obuf.at[(bs - 1) % 2], out.at[oidx[bs]],
                outsem.at[(bs - 1) % 2]).wait()

    return _body


def _mix_x(xr):
    bs, r, _ = xr.shape
    ld, oidx, e, hd, cs = _schedule(bs)
    grid_spec = pltpu.PrefetchScalarGridSpec(
        num_scalar_prefetch=5,
        grid=(bs + 1,),
        in_specs=[pl.BlockSpec(memory_space=pl.ANY)],
        out_specs=pl.BlockSpec(memory_space=pl.ANY),
        scratch_shapes=[
            pltpu.VMEM((_NBUF, r, 128), jnp.float32),
            pltpu.VMEM((2, r, 128), jnp.float32),
            pltpu.VMEM((r, 128), jnp.float32),
            pltpu.SemaphoreType.DMA((_NBUF,)),
            pltpu.SemaphoreType.DMA((2,)),
        ],
    )
    return pl.pallas_call(
        _make_tc_body(bs),
        grid_spec=grid_spec,
        out_shape=jax.ShapeDtypeStruct((bs, r, 128), jnp.float32),
        compiler_params=pltpu.CompilerParams(
            dimension_semantics=("arbitrary",),
        ),
    )(jnp.asarray(ld), jnp.asarray(oidx), jnp.asarray(e), jnp.asarray(hd),
      jnp.asarray(cs), xr)


def _sc_label_body(y2_hbm, ya_hbm, pidx_hbm, c2_hbm, yo_hbm, zo_hbm,
                   idx_v, cv, y2s, y2p, zas, zap, yov, zov, sem):
    ncp = y2s.shape[1]
    wid = lax.axis_index("s") * 2 + lax.axis_index("c")
    rows = idx_v.shape[0]
    base = wid * rows
    pltpu.sync_copy(pidx_hbm.at[pl.ds(base, rows)], idx_v)
    pltpu.sync_copy(c2_hbm.at[pl.ds(base, rows)], cv)
    pltpu.sync_copy(y2_hbm.at[pl.ds(base, rows)], y2s)
    pltpu.sync_copy(ya_hbm.at[pl.ds(base, rows)], zas)
    pltpu.async_copy(y2_hbm.at[idx_v], y2p, sem).wait()
    pltpu.async_copy(ya_hbm.at[idx_v], zap, sem).wait()
    for i in range(rows):
        cvec = cv[i]

        def chunk(j, carry, i=i, cvec=cvec):
            sl = pl.ds(j * 16, 16)
            a = y2s[i, sl]
            b = y2p[i, sl]
            ym = b + cvec * (a - b)
            yov[i, sl] = ym
            z = jnp.maximum(zas[i, sl], zap[i, sl]) - ym
            zov[i, sl] = jnp.minimum(jnp.maximum(z, 0.0), 1.0)
            return carry

        lax.fori_loop(0, ncp // 16, chunk, 0)
    pltpu.sync_copy(yov, yo_hbm.at[pl.ds(base, rows)])
    pltpu.sync_copy(zov, zo_hbm.at[pl.ds(base, rows)])


def _mix_labels(y2, ya2, perm, coeffs):
    bs, ncp = y2.shape
    nw = 32
    rows = bs // nw
    mesh = plsc.VectorSubcoreMesh(core_axis_name="c", subcore_axis_name="s")
    f32 = jnp.float32
    k = functools.partial(
        pl.kernel,
        mesh=mesh,
        out_type=[
            jax.ShapeDtypeStruct((bs, ncp), f32),
            jax.ShapeDtypeStruct((bs, ncp), f32),
        ],
        scratch_types=[
            pltpu.VMEM((rows,), jnp.int32),
            pltpu.VMEM((rows, 16), f32),
            pltpu.VMEM((rows, ncp), f32),
            pltpu.VMEM((rows, ncp), f32),
            pltpu.VMEM((rows, ncp), f32),
            pltpu.VMEM((rows, ncp), f32),
            pltpu.VMEM((rows, ncp), f32),
            pltpu.VMEM((rows, ncp), f32),
            pltpu.SemaphoreType.DMA,
        ],
    )(_sc_label_body)
    c2 = jnp.broadcast_to(coeffs[:, None], (bs, 16))
    return k(y2, ya2, perm, c2)


def kernel(x, y, y_aux, w):
    bs = x.shape[0]
    n = int(np.prod(x.shape[1:]))
    assert n % 128 == 0
    xr = x.reshape(bs, n // 128, 128)

    nc = y.shape[1]
    pad = (-(nc + 1)) % 1024
    y2 = jnp.concatenate(
        [y, w[:, None], jnp.zeros((bs, pad), jnp.float32)], axis=1)
    ncp = nc + 1 + pad
    ya2 = jnp.pad(y_aux, ((0, 0), (0, ncp - nc)))

    perm, coeffs = _mix_constants(bs)
    yo, zo = _mix_labels(y2, ya2, jnp.asarray(perm), jnp.asarray(coeffs))
    xo = _mix_x(xr)

    x_mix = xo.reshape(x.shape)
    y_mix = yo[:, :nc]
    w_mix = yo[:, nc]
    ya_mix = zo[:, :nc]
    return (x_mix, y_mix, ya_mix, w_mix)
